# Initial kernel scaffold; baseline (speedup 1.0000x reference)
import jax
import jax.numpy as jnp
from jax.experimental import pallas as pl


def kernel(x_0, incidence_1, W, b):
    raise NotImplementedError("write your pallas kernel here")


# speedup vs baseline: 1.1579x; 1.1579x over previous
import jax
import jax.numpy as jnp
from jax.experimental import pallas as pl
from jax.experimental.pallas import tpu as pltpu

_N = 4096
_E = 4096
_D = 128
_BK = 256
_NB = _N // _BK


def _body(x0_ref, inc_ref, w_ref, b_ref, out_ref,
          xhl_s, m_s, b16_s, mb_s):
    p = pl.program_id(0)
    i = pl.program_id(1)

    @pl.when(jnp.logical_and(p == 0, i == 0))
    def _init():
        x = jax.lax.dot_general(
            x0_ref[...], w_ref[...],
            dimension_numbers=(((1,), (1,)), ((), ())),
            preferred_element_type=jnp.float32,
            precision=jax.lax.Precision.HIGHEST,
        ) + b_ref[...]
        x_hi = x.astype(jnp.bfloat16)
        x_lo = (x - x_hi.astype(jnp.float32)).astype(jnp.bfloat16)
        xhl_s[:, :_D] = x_hi
        xhl_s[:, _D:] = x_lo
        m_s[...] = jnp.zeros_like(m_s)

    @pl.when(p == 0)
    def _phase0():
        blk16 = inc_ref[...].astype(jnp.bfloat16)
        b16_s[pl.ds(i * _BK, _BK), :] = blk16
        part = jax.lax.dot_general(
            blk16, xhl_s[pl.ds(i * _BK, _BK), :],
            dimension_numbers=(((0,), (0,)), ((), ())),
            preferred_element_type=jnp.float32,
        )
        m_s[...] = m_s[...] + part[:, :_D] + part[:, _D:]
        out_ref[...] = jnp.zeros_like(out_ref)

    @pl.when(p == 1)
    def _phase1():
        @pl.when(i == 0)
        def _round_m():
            mb_s[...] = m_s[...].astype(jnp.bfloat16)

        blk16 = b16_s[pl.ds(i * _BK, _BK), :]
        deg = jnp.sum(blk16.astype(jnp.float32), axis=1, keepdims=True)
        denom = jnp.where(deg > 0.0, deg, 1.0)
        m1 = jax.lax.dot_general(
            blk16, mb_s[...],
            dimension_numbers=(((1,), (0,)), ((), ())),
            preferred_element_type=jnp.float32,
        )
        x_blk = (xhl_s[pl.ds(i * _BK, _BK), :_D].astype(jnp.float32)
                 + xhl_s[pl.ds(i * _BK, _BK), _D:].astype(jnp.float32))
        out_ref[...] = x_blk + m1 / denom


def kernel(x_0, incidence_1, W, b):
    b2 = b.reshape(1, _D)
    return pl.pallas_call(
        _body,
        grid=(2, _NB),
        in_specs=[
            pl.BlockSpec((_N, _D), lambda p, i: (0, 0)),
            pl.BlockSpec((_BK, _E), lambda p, i: (i, 0)),
            pl.BlockSpec((_D, _D), lambda p, i: (0, 0)),
            pl.BlockSpec((1, _D), lambda p, i: (0, 0)),
        ],
        out_specs=pl.BlockSpec((_BK, _D), lambda p, i: (i, 0)),
        out_shape=jax.ShapeDtypeStruct((_N, _D), jnp.float32),
        scratch_shapes=[
            pltpu.VMEM((_N, 2 * _D), jnp.bfloat16),
            pltpu.VMEM((_N, _D), jnp.float32),
            pltpu.VMEM((_N, _E), jnp.bfloat16),
            pltpu.VMEM((_N, _D), jnp.bfloat16),
        ],
        compiler_params=pltpu.CompilerParams(
            dimension_semantics=("arbitrary", "arbitrary"),
        ),
    )(x_0, incidence_1, W, b2)


# speedup vs baseline: 1.3524x; 1.1680x over previous
import jax
import jax.numpy as jnp
from jax.experimental import pallas as pl
from jax.experimental.pallas import tpu as pltpu

_N = 4096
_E = 4096
_D = 128
_BK = 512
_NB = _N // _BK


def _body(x0_ref, inc_ref, w_ref, b_ref, out_ref,
          xhl_s, m_s, b16_s, mb_s):
    p = pl.program_id(0)
    i = pl.program_id(1)

    @pl.when(p == 0)
    def _phase0():
        x = jax.lax.dot_general(
            x0_ref[...], w_ref[...],
            dimension_numbers=(((1,), (1,)), ((), ())),
            preferred_element_type=jnp.float32,
            precision=jax.lax.Precision.HIGHEST,
        ) + b_ref[...]
        x_hi = x.astype(jnp.bfloat16)
        x_lo = (x - x_hi.astype(jnp.float32)).astype(jnp.bfloat16)
        xhl = jnp.concatenate([x_hi, x_lo], axis=1)
        xhl_s[pl.ds(i * _BK, _BK), :] = xhl

        blk16 = inc_ref[...].astype(jnp.bfloat16)
        b16_s[pl.ds(i * _BK, _BK), :] = blk16
        part = jax.lax.dot_general(
            blk16, xhl,
            dimension_numbers=(((0,), (0,)), ((), ())),
            preferred_element_type=jnp.float32,
        )
        acc = part[:, :_D] + part[:, _D:]

        @pl.when(i == 0)
        def _first():
            m_s[...] = acc

        @pl.when(i > 0)
        def _rest():
            m_s[...] = m_s[...] + acc

        out_ref[...] = jnp.zeros_like(out_ref)

    @pl.when(p == 1)
    def _phase1():
        @pl.when(i == 0)
        def _round_m():
            mb_s[...] = m_s[...].astype(jnp.bfloat16)

        blk16 = b16_s[pl.ds(i * _BK, _BK), :]
        deg = jnp.sum(blk16.astype(jnp.float32), axis=1, keepdims=True)
        denom = jnp.where(deg > 0.0, deg, 1.0)
        m1 = jax.lax.dot_general(
            blk16, mb_s[...],
            dimension_numbers=(((1,), (0,)), ((), ())),
            preferred_element_type=jnp.float32,
        )
        x_blk = (xhl_s[pl.ds(i * _BK, _BK), :_D].astype(jnp.float32)
                 + xhl_s[pl.ds(i * _BK, _BK), _D:].astype(jnp.float32))
        out_ref[...] = x_blk + m1 / denom


def kernel(x_0, incidence_1, W, b):
    b2 = b.reshape(1, _D)
    return pl.pallas_call(
        _body,
        grid=(2, _NB),
        in_specs=[
            pl.BlockSpec((_BK, _D), lambda p, i: (i, 0)),
            pl.BlockSpec((_BK, _E), lambda p, i: (i, 0)),
            pl.BlockSpec((_D, _D), lambda p, i: (0, 0)),
            pl.BlockSpec((1, _D), lambda p, i: (0, 0)),
        ],
        out_specs=pl.BlockSpec((_BK, _D), lambda p, i: (i, 0)),
        out_shape=jax.ShapeDtypeStruct((_N, _D), jnp.float32),
        scratch_shapes=[
            pltpu.VMEM((_N, 2 * _D), jnp.bfloat16),
            pltpu.VMEM((_N, _D), jnp.float32),
            pltpu.VMEM((_N, _E), jnp.bfloat16),
            pltpu.VMEM((_N, _D), jnp.bfloat16),
        ],
        compiler_params=pltpu.CompilerParams(
            dimension_semantics=("arbitrary", "arbitrary"),
        ),
    )(x_0, incidence_1, W, b2)


# speedup vs baseline: 1.6592x; 1.2269x over previous
import jax
import jax.numpy as jnp
from jax.experimental import pallas as pl
from jax.experimental.pallas import tpu as pltpu

_N = 4096
_E = 4096
_D = 128
_BK = 512
_NB = _N // _BK


def _body(x0_ref, inc_ref, w_ref, b_ref, out_ref,
          xhl_s, m_s, b16_s, mb_s):
    p = pl.program_id(0)
    i = pl.program_id(1)

    @pl.when(p == 0)
    def _phase0():
        x = jax.lax.dot_general(
            x0_ref[...], w_ref[...],
            dimension_numbers=(((1,), (1,)), ((), ())),
            preferred_element_type=jnp.float32,
            precision=jax.lax.Precision.HIGHEST,
        ) + b_ref[...]
        x_hi = x.astype(jnp.bfloat16)
        x_lo = (x - x_hi.astype(jnp.float32)).astype(jnp.bfloat16)
        xhl = jnp.concatenate([x_hi, x_lo], axis=1)
        xhl_s[pl.ds(i * _BK, _BK), :] = xhl

        blk16 = inc_ref[...].astype(jnp.bfloat16)
        b16_s[pl.ds(i * _BK, _BK), :] = blk16
        part = jax.lax.dot_general(
            blk16, xhl,
            dimension_numbers=(((0,), (0,)), ((), ())),
            preferred_element_type=jnp.float32,
        )
        acc = part[:, :_D] + part[:, _D:]

        @pl.when(i == 0)
        def _first():
            m_s[...] = acc

        @pl.when(i > 0)
        def _rest():
            m_s[...] = m_s[...] + acc

    @pl.when(p == 1)
    def _phase1():
        @pl.when(i == 0)
        def _round_m():
            mb_s[...] = m_s[...].astype(jnp.bfloat16)

        blk16 = b16_s[pl.ds(i * _BK, _BK), :]
        deg = jnp.sum(blk16.astype(jnp.float32), axis=1, keepdims=True)
        denom = jnp.where(deg > 0.0, deg, 1.0)
        m1 = jax.lax.dot_general(
            blk16, mb_s[...],
            dimension_numbers=(((1,), (0,)), ((), ())),
            preferred_element_type=jnp.float32,
        )
        x_blk = (xhl_s[pl.ds(i * _BK, _BK), :_D].astype(jnp.float32)
                 + xhl_s[pl.ds(i * _BK, _BK), _D:].astype(jnp.float32))
        out_ref[...] = x_blk + m1 / denom


def _in_idx(p, i):
    return (jnp.where(p == 0, i, _NB - 1), 0)


def _out_idx(p, i):
    return (jnp.where(p == 0, 0, i), 0)


def kernel(x_0, incidence_1, W, b):
    b2 = b.reshape(1, _D)
    return pl.pallas_call(
        _body,
        grid=(2, _NB),
        in_specs=[
            pl.BlockSpec((_BK, _D), _in_idx),
            pl.BlockSpec((_BK, _E), _in_idx),
            pl.BlockSpec((_D, _D), lambda p, i: (0, 0)),
            pl.BlockSpec((1, _D), lambda p, i: (0, 0)),
        ],
        out_specs=pl.BlockSpec((_BK, _D), _out_idx),
        out_shape=jax.ShapeDtypeStruct((_N, _D), jnp.float32),
        scratch_shapes=[
            pltpu.VMEM((_N, 2 * _D), jnp.bfloat16),
            pltpu.VMEM((_N, _D), jnp.float32),
            pltpu.VMEM((_N, _E), jnp.bfloat16),
            pltpu.VMEM((_N, _D), jnp.bfloat16),
        ],
        compiler_params=pltpu.CompilerParams(
            dimension_semantics=("arbitrary", "arbitrary"),
        ),
    )(x_0, incidence_1, W, b2)


# speedup vs baseline: 1.7685x; 1.0659x over previous
import jax
import jax.numpy as jnp
from jax.experimental import pallas as pl
from jax.experimental.pallas import tpu as pltpu

_N = 4096
_E = 4096
_D = 128
_BK = 512
_NB = _N // _BK


def _body(x0_ref, inc_ref, w_ref, b_ref, out_ref,
          xhl_s, m_s, b16_s, mb_s, r_s):
    p = pl.program_id(0)
    i = pl.program_id(1)

    @pl.when(p == 0)
    def _phase0():
        x = jax.lax.dot_general(
            x0_ref[...], w_ref[...],
            dimension_numbers=(((1,), (1,)), ((), ())),
            preferred_element_type=jnp.float32,
            precision=jax.lax.Precision.HIGHEST,
        ) + b_ref[...]
        x_hi = x.astype(jnp.bfloat16)
        x_lo = (x - x_hi.astype(jnp.float32)).astype(jnp.bfloat16)
        xhl = jnp.concatenate([x_hi, x_lo], axis=1)
        xhl_s[pl.ds(i * _BK, _BK), :] = xhl

        blk = inc_ref[...]
        deg = jnp.sum(blk, axis=1, keepdims=True)
        r_s[pl.ds(i * _BK, _BK), :] = 1.0 / jnp.maximum(deg, 1.0)

        blk16 = blk.astype(jnp.bfloat16)
        b16_s[pl.ds(i * _BK, _BK), :] = blk16
        part = jax.lax.dot_general(
            blk16, xhl,
            dimension_numbers=(((0,), (0,)), ((), ())),
            preferred_element_type=jnp.float32,
        )
        acc = part[:, :_D] + part[:, _D:]

        @pl.when(i == 0)
        def _first():
            m_s[...] = acc

        @pl.when(i > 0)
        def _rest():
            m_s[...] = m_s[...] + acc

    @pl.when(p == 1)
    def _phase1():
        @pl.when(i == 0)
        def _round_m():
            mb_s[...] = m_s[...].astype(jnp.bfloat16)

        blk16 = b16_s[pl.ds(i * _BK, _BK), :]
        m1 = jax.lax.dot_general(
            blk16, mb_s[...],
            dimension_numbers=(((1,), (0,)), ((), ())),
            preferred_element_type=jnp.float32,
        )
        x_blk = (xhl_s[pl.ds(i * _BK, _BK), :_D].astype(jnp.float32)
                 + xhl_s[pl.ds(i * _BK, _BK), _D:].astype(jnp.float32))
        out_ref[...] = x_blk + m1 * r_s[pl.ds(i * _BK, _BK), :]


def _in_idx(p, i):
    return (jnp.where(p == 0, i, _NB - 1), 0)


def _out_idx(p, i):
    return (jnp.where(p == 0, 0, i), 0)


def kernel(x_0, incidence_1, W, b):
    b2 = b.reshape(1, _D)
    return pl.pallas_call(
        _body,
        grid=(2, _NB),
        in_specs=[
            pl.BlockSpec((_BK, _D), _in_idx),
            pl.BlockSpec((_BK, _E), _in_idx),
            pl.BlockSpec((_D, _D), lambda p, i: (0, 0)),
            pl.BlockSpec((1, _D), lambda p, i: (0, 0)),
        ],
        out_specs=pl.BlockSpec((_BK, _D), _out_idx),
        out_shape=jax.ShapeDtypeStruct((_N, _D), jnp.float32),
        scratch_shapes=[
            pltpu.VMEM((_N, 2 * _D), jnp.bfloat16),
            pltpu.VMEM((_N, _D), jnp.float32),
            pltpu.VMEM((_N, _E), jnp.bfloat16),
            pltpu.VMEM((_N, _D), jnp.bfloat16),
            pltpu.VMEM((_N, 1), jnp.float32),
        ],
        compiler_params=pltpu.CompilerParams(
            dimension_semantics=("arbitrary", "arbitrary"),
        ),
    )(x_0, incidence_1, W, b2)
